# baseline (speedup 1.0000x reference)
import functools

import jax
import jax.numpy as jnp
from jax import lax
from jax.experimental.layout import Format, Layout
from jax.experimental import pallas as pl
from jax.experimental.pallas import tpu as pltpu
from jax.experimental.pallas import tpu_sc as plsc

V = 1024
D = 1024
B = 4096 * 50
NC, NS = 2, 16
NW = NS
BPW = B // NW
C = 16
NBUF = 4
NCH = BPW // C


def _emb_body(idx_hbm, table_hbm, out_hbm, idx_v, rows, sg, ss):
    wid = lax.axis_index("s")
    base = wid * BPW
    pltpu.sync_copy(idx_hbm.at[wid], idx_v)

    def gather(j, b):
        pltpu.async_copy(
            table_hbm.at[idx_v.at[pl.ds(j * C, C)]], rows[b], sg[b])

    def wait_gather(j, b):
        pltpu.make_async_copy(
            table_hbm.at[idx_v.at[pl.ds(j * C, C)]], rows[b], sg[b]).wait()

    def scatter(j, b):
        pltpu.async_copy(rows[b], out_hbm.at[pl.ds(base + j * C, C)], ss[b])

    def wait_scatter(b):
        pltpu.make_async_copy(rows[b], out_hbm.at[pl.ds(base, C)],
                              ss[b]).wait()

    for b in range(NBUF):
        gather(b, b)

    def body(i, carry):
        j0 = i * NBUF
        for b in range(NBUF):
            wait_gather(j0 + b, b)
            scatter(j0 + b, b)
        for b in range(NBUF):
            @pl.when(j0 + b + NBUF < NCH)
            def _(b=b):
                wait_scatter(b)
                gather(j0 + b + NBUF, b)
        return carry

    lax.fori_loop(0, NCH // NBUF, body, 0, unroll=False)
    for b in range(NBUF):
        wait_scatter(b)


def _jit_kernel():
    sharding = jax.sharding.SingleDeviceSharding(jax.devices()[0])
    fmt = Format(Layout(major_to_minor=(0, 1, 2), tiling=((16,),)), sharding)
    return jax.jit(_kernel_impl, out_shardings=fmt)


_cached = None


def kernel(indices, emb_weight):
    global _cached
    if _cached is None:
        _cached = _jit_kernel()
    return _cached(indices, emb_weight)


def _kernel_impl(indices, emb_weight):
    idx = indices.reshape(NW, BPW).astype(jnp.int32)
    mesh = plsc.VectorSubcoreMesh(
        core_axis_name="c", subcore_axis_name="s", num_cores=1)

    def wrapped(idx_hbm, table_hbm, out_hbm, idx_v, r0, r1, r2, r3,
                g0, g1, g2, g3, s0, s1, s2, s3):
        _emb_body(idx_hbm, table_hbm, out_hbm, idx_v,
                  [r0, r1, r2, r3], [g0, g1, g2, g3], [s0, s1, s2, s3])

    fn = pl.kernel(
        wrapped,
        out_type=jax.ShapeDtypeStruct((B, D), jnp.float32),
        mesh=mesh,
        scratch_types=[
            pltpu.VMEM((BPW,), jnp.int32),
            pltpu.VMEM((C, D), jnp.float32),
            pltpu.VMEM((C, D), jnp.float32),
            pltpu.VMEM((C, D), jnp.float32),
            pltpu.VMEM((C, D), jnp.float32),
            pltpu.SemaphoreType.DMA,
            pltpu.SemaphoreType.DMA,
            pltpu.SemaphoreType.DMA,
            pltpu.SemaphoreType.DMA,
            pltpu.SemaphoreType.DMA,
            pltpu.SemaphoreType.DMA,
            pltpu.SemaphoreType.DMA,
            pltpu.SemaphoreType.DMA,
        ],
    )
    out = fn(idx, emb_weight)
    return out.reshape(4096, 50, D)


# speedup vs baseline: 1.0444x; 1.0444x over previous
import functools

import jax
import jax.numpy as jnp
from jax import lax
from jax.experimental import pallas as pl
from jax.experimental.pallas import tpu as pltpu
from jax.experimental.pallas import tpu_sc as plsc

V = 1024
D = 1024
B = 4096 * 50

SC_B = 112640
NC, NS = 2, 16
NW = NC * NS
BPW = SC_B // NW
C = 16
NBUF = 4
NCH = BPW // C

M = 512
TC_B = B - SC_B


def _sc_body(idx_hbm, table_hbm, out_hbm, idx_v, rows, sg, ss):
    wid = lax.axis_index("s") * NC + lax.axis_index("c")
    base = wid * BPW
    pltpu.sync_copy(idx_hbm.at[wid], idx_v)

    def gather(j, b):
        pltpu.async_copy(
            table_hbm.at[idx_v.at[pl.ds(j * C, C)]], rows[b], sg[b])

    def wait_gather(j, b):
        pltpu.make_async_copy(
            table_hbm.at[idx_v.at[pl.ds(j * C, C)]], rows[b], sg[b]).wait()

    def scatter(j, b):
        pltpu.async_copy(rows[b], out_hbm.at[pl.ds(base + j * C, C)], ss[b])

    def wait_scatter(b):
        pltpu.make_async_copy(rows[b], out_hbm.at[pl.ds(base, C)],
                              ss[b]).wait()

    for b in range(NBUF):
        gather(b, b)

    def body(i, carry):
        j0 = i * NBUF
        for b in range(NBUF):
            wait_gather(j0 + b, b)
            scatter(j0 + b, b)
        for b in range(NBUF):
            @pl.when(j0 + b + NBUF < NCH)
            def _(b=b):
                wait_scatter(b)
                gather(j0 + b + NBUF, b)
        return carry

    lax.fori_loop(0, NCH // NBUF, body, 0, unroll=False)
    for b in range(NBUF):
        wait_scatter(b)


def _sc_call(idx, table):
    mesh = plsc.VectorSubcoreMesh(core_axis_name="c", subcore_axis_name="s")

    def wrapped(idx_hbm, table_hbm, out_hbm, idx_v, r0, r1, r2, r3,
                g0, g1, g2, g3, s0, s1, s2, s3):
        _sc_body(idx_hbm, table_hbm, out_hbm, idx_v,
                 [r0, r1, r2, r3], [g0, g1, g2, g3], [s0, s1, s2, s3])

    fn = pl.kernel(
        wrapped,
        out_type=jax.ShapeDtypeStruct((B, D), jnp.float32),
        mesh=mesh,
        scratch_types=[
            pltpu.VMEM((BPW,), jnp.int32),
            pltpu.VMEM((C, D), jnp.float32),
            pltpu.VMEM((C, D), jnp.float32),
            pltpu.VMEM((C, D), jnp.float32),
            pltpu.VMEM((C, D), jnp.float32),
            pltpu.SemaphoreType.DMA,
            pltpu.SemaphoreType.DMA,
            pltpu.SemaphoreType.DMA,
            pltpu.SemaphoreType.DMA,
            pltpu.SemaphoreType.DMA,
            pltpu.SemaphoreType.DMA,
            pltpu.SemaphoreType.DMA,
            pltpu.SemaphoreType.DMA,
        ],
    )
    return fn(idx, table)


def _tc_body(idx_ref, hi_ref, lo_ref, out_ref):
    idxv = idx_ref[0, 0, :]
    iota = lax.broadcasted_iota(jnp.int32, (M, V), 1)
    oh = (idxv[:, None] == iota).astype(jnp.bfloat16)
    acc = jnp.dot(oh, hi_ref[...], preferred_element_type=jnp.float32)
    acc = acc + jnp.dot(oh, lo_ref[...], preferred_element_type=jnp.float32)
    out_ref[...] = acc


def _tc_call(idx, table):
    hi = table.astype(jnp.bfloat16)
    lo = (table - hi.astype(jnp.float32)).astype(jnp.bfloat16)
    return pl.pallas_call(
        _tc_body,
        grid=(TC_B // M,),
        in_specs=[
            pl.BlockSpec((1, 1, M), lambda i: (i, 0, 0)),
            pl.BlockSpec((V, D), lambda i: (0, 0)),
            pl.BlockSpec((V, D), lambda i: (0, 0)),
        ],
        out_specs=pl.BlockSpec((M, D), lambda i: (i, 0)),
        out_shape=jax.ShapeDtypeStruct((TC_B, D), jnp.float32),
    )(idx.reshape(TC_B // M, 1, M), hi, lo)


@jax.jit
def kernel(indices, emb_weight):
    idx = indices.reshape(B).astype(jnp.int32)
    sc_full = _sc_call(idx[:SC_B].reshape(NW, BPW), emb_weight)
    tc_out = _tc_call(idx[SC_B:], emb_weight)
    out = lax.dynamic_update_slice(sc_full, tc_out, (SC_B, 0))
    return out.reshape(4096, 50, D)


# speedup vs baseline: 1.1317x; 1.0835x over previous
import functools

import jax
import jax.numpy as jnp
from jax import lax
from jax.experimental import pallas as pl
from jax.experimental.pallas import tpu as pltpu
from jax.experimental.pallas import tpu_sc as plsc

V = 1024
D = 1024
B = 4096 * 50
NC, NS = 2, 16
NW = NC * NS
BPW = B // NW
C = 16
NBUF = 4
NCH = BPW // C


def _emb_body(idx_hbm, table_hbm, out_hbm, idx_v, rows, sg, ss):
    wid = lax.axis_index("s") * NC + lax.axis_index("c")
    base = wid * BPW
    pltpu.sync_copy(idx_hbm.at[wid], idx_v)

    def gather(j, b):
        pltpu.async_copy(
            table_hbm.at[idx_v.at[pl.ds(j * C, C)]], rows[b], sg[b])

    def wait_gather(j, b):
        pltpu.make_async_copy(
            table_hbm.at[idx_v.at[pl.ds(j * C, C)]], rows[b], sg[b]).wait()

    def scatter(j, b):
        pltpu.async_copy(rows[b], out_hbm.at[pl.ds(base + j * C, C)], ss[b])

    def wait_scatter(b):
        pltpu.make_async_copy(rows[b], out_hbm.at[pl.ds(base, C)],
                              ss[b]).wait()

    for b in range(NBUF):
        gather(b, b)

    def body(i, carry):
        j0 = i * NBUF
        for b in range(NBUF):
            wait_gather(j0 + b, b)
            scatter(j0 + b, b)
        for b in range(NBUF):
            @pl.when(j0 + b + NBUF < NCH)
            def _(b=b):
                wait_scatter(b)
                gather(j0 + b + NBUF, b)
        return carry

    lax.fori_loop(0, NCH // NBUF, body, 0, unroll=False)
    for b in range(NBUF):
        wait_scatter(b)


@jax.jit
def kernel(indices, emb_weight):
    idx = indices.reshape(NW, BPW).astype(jnp.int32)
    mesh = plsc.VectorSubcoreMesh(core_axis_name="c", subcore_axis_name="s")

    def wrapped(idx_hbm, table_hbm, out_hbm, idx_v, r0, r1, r2, r3,
                g0, g1, g2, g3, s0, s1, s2, s3):
        _emb_body(idx_hbm, table_hbm, out_hbm, idx_v,
                  [r0, r1, r2, r3], [g0, g1, g2, g3], [s0, s1, s2, s3])

    fn = pl.kernel(
        wrapped,
        out_type=jax.ShapeDtypeStruct((B, D), jnp.float32),
        mesh=mesh,
        scratch_types=[
            pltpu.VMEM((BPW,), jnp.int32),
            pltpu.VMEM((C, D), jnp.float32),
            pltpu.VMEM((C, D), jnp.float32),
            pltpu.VMEM((C, D), jnp.float32),
            pltpu.VMEM((C, D), jnp.float32),
            pltpu.SemaphoreType.DMA,
            pltpu.SemaphoreType.DMA,
            pltpu.SemaphoreType.DMA,
            pltpu.SemaphoreType.DMA,
            pltpu.SemaphoreType.DMA,
            pltpu.SemaphoreType.DMA,
            pltpu.SemaphoreType.DMA,
            pltpu.SemaphoreType.DMA,
        ],
    )
    out = fn(idx, emb_weight)
    return out.reshape(4096, 50, D)
